# Initial kernel scaffold; baseline (speedup 1.0000x reference)
import jax
import jax.numpy as jnp
from jax.experimental import pallas as pl


def kernel(x, cdk_desc, edge_index, W0a, b0a, W0b, b0b, g0, be0, W1a, b1a, W1b, b1b, g1, be1, Wlin, blin):
    raise NotImplementedError("write your pallas kernel here")


# speedup vs baseline: 6.1276x; 6.1276x over previous
import functools

import jax
import jax.numpy as jnp
from jax import lax
from jax.experimental import pallas as pl
from jax.experimental.pallas import tpu as pltpu
from jax.experimental.pallas import tpu_sc as plsc

N_NODES = 10000
N_EDGES = 320000
EDGE_CHUNK = 80
NUM_WORKERS = 32
SUBCORES = 16


def _sc_segment_add(h, src2, dst2, zeros):
    n, f = h.shape
    nblk = src2.shape[0] // NUM_WORKERS
    sb = 25
    nsup = nblk // sb
    rz = n // SUBCORES
    mesh = plsc.VectorSubcoreMesh(core_axis_name="c", subcore_axis_name="s")

    @functools.partial(
        pl.kernel,
        mesh=mesh,
        out_type=jax.ShapeDtypeStruct((2, n, f), jnp.float32),
        compiler_params=pltpu.CompilerParams(use_tc_tiling_on_sc=False),
        scratch_types=[
            pltpu.VMEM_SHARED((n, f), jnp.float32),
            pltpu.VMEM((sb, EDGE_CHUNK), jnp.int32),
            pltpu.VMEM((sb, EDGE_CHUNK), jnp.int32),
            pltpu.VMEM((EDGE_CHUNK, f), jnp.float32),
            pltpu.SemaphoreType.DMA,
        ],
    )
    def k(h_hbm, src_hbm, dst_hbm, z_hbm, out_hbm, acc, src_v, dst_v, rows_v, sem):
        c = lax.axis_index("c")
        s = lax.axis_index("s")
        wid = c * SUBCORES + s
        pltpu.sync_copy(z_hbm.at[pl.ds(s * rz, rz)], acc.at[pl.ds(s * rz, rz)])
        plsc.subcore_barrier()

        @pl.loop(0, nsup)
        def _(t):
            base = wid * nblk + t * sb
            pltpu.sync_copy(src_hbm.at[pl.ds(base, sb)], src_v)
            pltpu.sync_copy(dst_hbm.at[pl.ds(base, sb)], dst_v)

            @pl.loop(0, sb)
            def _(j):
                pltpu.async_copy(h_hbm.at[src_v.at[j]], rows_v, sem).wait()
                pltpu.sync_copy(rows_v, acc.at[dst_v.at[j]], add=True)

        plsc.subcore_barrier()
        pltpu.sync_copy(acc.at[pl.ds(s * rz, rz)], out_hbm.at[c, pl.ds(s * rz, rz)])

    return k(h, src2, dst2, zeros)


def _mlp_stats(h, agg, Wa, ba, Wb, bb):
    n, f = h.shape
    hdim = Wb.shape[1]
    blk = 1000
    grid = n // blk

    def body(h_ref, p0_ref, p1_ref, wa_ref, ba_ref, wb_ref, bb_ref,
             m_ref, st_ref, acc):
        i = pl.program_id(0)
        m = h_ref[...] + p0_ref[0] + p1_ref[0]
        t = jnp.maximum(
            jnp.dot(m, wa_ref[...], preferred_element_type=jnp.float32)
            + ba_ref[...], 0.0)
        m2 = (jnp.dot(t, wb_ref[...], preferred_element_type=jnp.float32)
              + bb_ref[...])
        m_ref[...] = m2

        @pl.when(i == 0)
        def _():
            acc[...] = jnp.zeros_like(acc)

        acc[0:1, :] += jnp.sum(m2, axis=0, keepdims=True)
        acc[1:2, :] += jnp.sum(m2 * m2, axis=0, keepdims=True)

        @pl.when(i == grid - 1)
        def _():
            st_ref[...] = acc[0:2, :]

    return pl.pallas_call(
        body,
        grid=(grid,),
        in_specs=[
            pl.BlockSpec((blk, f), lambda i: (i, 0)),
            pl.BlockSpec((1, blk, f), lambda i: (0, i, 0)),
            pl.BlockSpec((1, blk, f), lambda i: (1, i, 0)),
            pl.BlockSpec((f, hdim), lambda i: (0, 0)),
            pl.BlockSpec((1, hdim), lambda i: (0, 0)),
            pl.BlockSpec((hdim, hdim), lambda i: (0, 0)),
            pl.BlockSpec((1, hdim), lambda i: (0, 0)),
        ],
        out_specs=[
            pl.BlockSpec((blk, hdim), lambda i: (i, 0)),
            pl.BlockSpec((2, hdim), lambda i: (0, 0)),
        ],
        out_shape=[
            jax.ShapeDtypeStruct((n, hdim), jnp.float32),
            jax.ShapeDtypeStruct((2, hdim), jnp.float32),
        ],
        scratch_shapes=[pltpu.VMEM((8, hdim), jnp.float32)],
    )(h, agg, agg, Wa, ba.reshape(1, -1), Wb, bb.reshape(1, -1))


def _bn_elu(m, st, g, be):
    n, hdim = m.shape
    blk = 1000
    grid = n // blk

    def body(m_ref, st_ref, g_ref, be_ref, o_ref):
        mean = st_ref[0:1, :] * (1.0 / n)
        var = st_ref[1:2, :] * (1.0 / n) - mean * mean
        scale = g_ref[...] * lax.rsqrt(var + 1e-5)
        shift = be_ref[...] - mean * scale
        v = m_ref[...] * scale + shift
        o_ref[...] = jnp.where(v > 0, v, jnp.exp(jnp.minimum(v, 0.0)) - 1.0)

    return pl.pallas_call(
        body,
        grid=(grid,),
        in_specs=[
            pl.BlockSpec((blk, hdim), lambda i: (i, 0)),
            pl.BlockSpec((2, hdim), lambda i: (0, 0)),
            pl.BlockSpec((1, hdim), lambda i: (0, 0)),
            pl.BlockSpec((1, hdim), lambda i: (0, 0)),
        ],
        out_specs=pl.BlockSpec((blk, hdim), lambda i: (i, 0)),
        out_shape=jax.ShapeDtypeStruct((n, hdim), jnp.float32),
    )(m, st, g.reshape(1, -1), be.reshape(1, -1))


def _bn_elu_head(m, st, g, be, Wlin, blin):
    n, hdim = m.shape
    blk = 1000
    grid = n // blk

    def body(m_ref, st_ref, g_ref, be_ref, w_ref, b_ref, o_ref):
        mean = st_ref[0:1, :] * (1.0 / n)
        var = st_ref[1:2, :] * (1.0 / n) - mean * mean
        scale = g_ref[...] * lax.rsqrt(var + 1e-5)
        shift = be_ref[...] - mean * scale
        v = m_ref[...] * scale + shift
        v = jnp.where(v > 0, v, jnp.exp(jnp.minimum(v, 0.0)) - 1.0)
        logit = jnp.sum(v * w_ref[...], axis=1, keepdims=True) + b_ref[...]
        o_ref[...] = 1.0 / (1.0 + jnp.exp(-logit))

    return pl.pallas_call(
        body,
        grid=(grid,),
        in_specs=[
            pl.BlockSpec((blk, hdim), lambda i: (i, 0)),
            pl.BlockSpec((2, hdim), lambda i: (0, 0)),
            pl.BlockSpec((1, hdim), lambda i: (0, 0)),
            pl.BlockSpec((1, hdim), lambda i: (0, 0)),
            pl.BlockSpec((1, hdim), lambda i: (0, 0)),
            pl.BlockSpec((1, 1), lambda i: (0, 0)),
        ],
        out_specs=pl.BlockSpec((blk, 1), lambda i: (i, 0)),
        out_shape=jax.ShapeDtypeStruct((n, 1), jnp.float32),
    )(m, st, g.reshape(1, -1), be.reshape(1, -1),
      Wlin.reshape(1, -1), blin.reshape(1, 1))


def kernel(x, cdk_desc, edge_index, W0a, b0a, W0b, b0b, g0, be0,
           W1a, b1a, W1b, b1b, g1, be1, Wlin, blin):
    h0 = jnp.concatenate([x, cdk_desc], axis=-1)
    src2 = edge_index[0].reshape(N_EDGES // EDGE_CHUNK, EDGE_CHUNK)
    dst2 = edge_index[1].reshape(N_EDGES // EDGE_CHUNK, EDGE_CHUNK)

    z160 = jnp.zeros((N_NODES, h0.shape[1]), jnp.float32)
    agg0 = _sc_segment_add(h0, src2, dst2, z160)
    m0, st0 = _mlp_stats(h0, agg0, W0a, b0a, W0b, b0b)
    h1 = _bn_elu(m0, st0, g0, be0)

    z128 = jnp.zeros((N_NODES, h1.shape[1]), jnp.float32)
    agg1 = _sc_segment_add(h1, src2, dst2, z128)
    m1, st1 = _mlp_stats(h1, agg1, W1a, b1a, W1b, b1b)
    out = _bn_elu_head(m1, st1, g1, be1, Wlin, blin)
    return out.reshape(-1)


# speedup vs baseline: 7.5229x; 1.2277x over previous
import functools

import jax
import jax.numpy as jnp
from jax import lax
from jax.experimental import pallas as pl
from jax.experimental.pallas import tpu as pltpu
from jax.experimental.pallas import tpu_sc as plsc

N_NODES = 10000
N_EDGES = 320000
EDGE_CHUNK = 80
NUM_WORKERS = 32
SUBCORES = 16


def _sc_segment_add(h, src2, dst2, zeros):
    n, f = h.shape
    nblk = src2.shape[0] // NUM_WORKERS
    sb = 25
    nsup = nblk // sb
    rz = n // SUBCORES
    mesh = plsc.VectorSubcoreMesh(core_axis_name="c", subcore_axis_name="s")

    @functools.partial(
        pl.kernel,
        mesh=mesh,
        out_type=jax.ShapeDtypeStruct((2, n, f), jnp.float32),
        compiler_params=pltpu.CompilerParams(use_tc_tiling_on_sc=False),
        scratch_types=[
            pltpu.VMEM_SHARED((n, f), jnp.float32),
            pltpu.VMEM((sb, EDGE_CHUNK), jnp.int32),
            pltpu.VMEM((sb, EDGE_CHUNK), jnp.int32),
            pltpu.VMEM((EDGE_CHUNK, f), jnp.float32),
            pltpu.VMEM((EDGE_CHUNK, f), jnp.float32),
            pltpu.SemaphoreType.DMA,
            pltpu.SemaphoreType.DMA,
            pltpu.SemaphoreType.DMA,
            pltpu.SemaphoreType.DMA,
        ],
    )
    def k(h_hbm, src_hbm, dst_hbm, z_hbm, out_hbm, acc,
          src_v, dst_v, rows0, rows1, g0, g1, s0, s1):
        c = lax.axis_index("c")
        s = lax.axis_index("s")
        wid = c * SUBCORES + s
        pltpu.sync_copy(z_hbm.at[pl.ds(s * rz, rz)], acc.at[pl.ds(s * rz, rz)])
        plsc.subcore_barrier()

        def start_g(j, buf, sem):
            pltpu.async_copy(h_hbm.at[src_v.at[j]], buf, sem)

        def wait_g(j, buf, sem):
            pltpu.make_async_copy(h_hbm.at[src_v.at[j]], buf, sem).wait()

        def start_s(j, buf, sem):
            pltpu.async_copy(buf, acc.at[dst_v.at[j]], sem, add=True)

        def wait_s(j, buf, sem):
            pltpu.make_async_copy(buf, acc.at[dst_v.at[j]], sem).wait()

        @pl.loop(0, nsup)
        def _(t):
            base = wid * nblk + t * sb
            pltpu.sync_copy(src_hbm.at[pl.ds(base, sb)], src_v)
            pltpu.sync_copy(dst_hbm.at[pl.ds(base, sb)], dst_v)
            start_g(0, rows0, g0)

            @pl.loop(0, (sb - 1) // 2)
            def _(p):
                j0 = 2 * p
                wait_g(j0, rows0, g0)
                start_g(j0 + 1, rows1, g1)
                start_s(j0, rows0, s0)
                wait_g(j0 + 1, rows1, g1)
                wait_s(j0, rows0, s0)
                start_g(j0 + 2, rows0, g0)
                start_s(j0 + 1, rows1, s1)
                wait_s(j0 + 1, rows1, s1)

            wait_g(sb - 1, rows0, g0)
            pltpu.sync_copy(rows0, acc.at[dst_v.at[sb - 1]], add=True)

        plsc.subcore_barrier()
        pltpu.sync_copy(acc.at[pl.ds(s * rz, rz)], out_hbm.at[c, pl.ds(s * rz, rz)])

    return k(h, src2, dst2, zeros)


def _mlp_stats(h, agg, Wa, ba, Wb, bb):
    n, f = h.shape
    hdim = Wb.shape[1]
    blk = 1000
    grid = n // blk

    def body(h_ref, p0_ref, p1_ref, wa_ref, ba_ref, wb_ref, bb_ref,
             m_ref, st_ref, acc):
        i = pl.program_id(0)
        m = h_ref[...] + p0_ref[0] + p1_ref[0]
        t = jnp.maximum(
            jnp.dot(m, wa_ref[...], preferred_element_type=jnp.float32)
            + ba_ref[...], 0.0)
        m2 = (jnp.dot(t, wb_ref[...], preferred_element_type=jnp.float32)
              + bb_ref[...])
        m_ref[...] = m2

        @pl.when(i == 0)
        def _():
            acc[...] = jnp.zeros_like(acc)

        acc[0:1, :] += jnp.sum(m2, axis=0, keepdims=True)
        acc[1:2, :] += jnp.sum(m2 * m2, axis=0, keepdims=True)

        @pl.when(i == grid - 1)
        def _():
            st_ref[...] = acc[0:2, :]

    return pl.pallas_call(
        body,
        grid=(grid,),
        in_specs=[
            pl.BlockSpec((blk, f), lambda i: (i, 0)),
            pl.BlockSpec((1, blk, f), lambda i: (0, i, 0)),
            pl.BlockSpec((1, blk, f), lambda i: (1, i, 0)),
            pl.BlockSpec((f, hdim), lambda i: (0, 0)),
            pl.BlockSpec((1, hdim), lambda i: (0, 0)),
            pl.BlockSpec((hdim, hdim), lambda i: (0, 0)),
            pl.BlockSpec((1, hdim), lambda i: (0, 0)),
        ],
        out_specs=[
            pl.BlockSpec((blk, hdim), lambda i: (i, 0)),
            pl.BlockSpec((2, hdim), lambda i: (0, 0)),
        ],
        out_shape=[
            jax.ShapeDtypeStruct((n, hdim), jnp.float32),
            jax.ShapeDtypeStruct((2, hdim), jnp.float32),
        ],
        scratch_shapes=[pltpu.VMEM((8, hdim), jnp.float32)],
    )(h, agg, agg, Wa, ba.reshape(1, -1), Wb, bb.reshape(1, -1))


def _bn_elu(m, st, g, be):
    n, hdim = m.shape
    blk = 1000
    grid = n // blk

    def body(m_ref, st_ref, g_ref, be_ref, o_ref):
        mean = st_ref[0:1, :] * (1.0 / n)
        var = st_ref[1:2, :] * (1.0 / n) - mean * mean
        scale = g_ref[...] * lax.rsqrt(var + 1e-5)
        shift = be_ref[...] - mean * scale
        v = m_ref[...] * scale + shift
        o_ref[...] = jnp.where(v > 0, v, jnp.exp(jnp.minimum(v, 0.0)) - 1.0)

    return pl.pallas_call(
        body,
        grid=(grid,),
        in_specs=[
            pl.BlockSpec((blk, hdim), lambda i: (i, 0)),
            pl.BlockSpec((2, hdim), lambda i: (0, 0)),
            pl.BlockSpec((1, hdim), lambda i: (0, 0)),
            pl.BlockSpec((1, hdim), lambda i: (0, 0)),
        ],
        out_specs=pl.BlockSpec((blk, hdim), lambda i: (i, 0)),
        out_shape=jax.ShapeDtypeStruct((n, hdim), jnp.float32),
    )(m, st, g.reshape(1, -1), be.reshape(1, -1))


def _bn_elu_head(m, st, g, be, Wlin, blin):
    n, hdim = m.shape
    blk = 1000
    grid = n // blk

    def body(m_ref, st_ref, g_ref, be_ref, w_ref, b_ref, o_ref):
        mean = st_ref[0:1, :] * (1.0 / n)
        var = st_ref[1:2, :] * (1.0 / n) - mean * mean
        scale = g_ref[...] * lax.rsqrt(var + 1e-5)
        shift = be_ref[...] - mean * scale
        v = m_ref[...] * scale + shift
        v = jnp.where(v > 0, v, jnp.exp(jnp.minimum(v, 0.0)) - 1.0)
        logit = jnp.sum(v * w_ref[...], axis=1, keepdims=True) + b_ref[...]
        o_ref[...] = 1.0 / (1.0 + jnp.exp(-logit))

    return pl.pallas_call(
        body,
        grid=(grid,),
        in_specs=[
            pl.BlockSpec((blk, hdim), lambda i: (i, 0)),
            pl.BlockSpec((2, hdim), lambda i: (0, 0)),
            pl.BlockSpec((1, hdim), lambda i: (0, 0)),
            pl.BlockSpec((1, hdim), lambda i: (0, 0)),
            pl.BlockSpec((1, hdim), lambda i: (0, 0)),
            pl.BlockSpec((1, 1), lambda i: (0, 0)),
        ],
        out_specs=pl.BlockSpec((blk, 1), lambda i: (i, 0)),
        out_shape=jax.ShapeDtypeStruct((n, 1), jnp.float32),
    )(m, st, g.reshape(1, -1), be.reshape(1, -1),
      Wlin.reshape(1, -1), blin.reshape(1, 1))


def kernel(x, cdk_desc, edge_index, W0a, b0a, W0b, b0b, g0, be0,
           W1a, b1a, W1b, b1b, g1, be1, Wlin, blin):
    h0 = jnp.concatenate([x, cdk_desc], axis=-1)
    src2 = edge_index[0].reshape(N_EDGES // EDGE_CHUNK, EDGE_CHUNK)
    dst2 = edge_index[1].reshape(N_EDGES // EDGE_CHUNK, EDGE_CHUNK)

    z160 = jnp.zeros((N_NODES, h0.shape[1]), jnp.float32)
    agg0 = _sc_segment_add(h0, src2, dst2, z160)
    m0, st0 = _mlp_stats(h0, agg0, W0a, b0a, W0b, b0b)
    h1 = _bn_elu(m0, st0, g0, be0)

    z128 = jnp.zeros((N_NODES, h1.shape[1]), jnp.float32)
    agg1 = _sc_segment_add(h1, src2, dst2, z128)
    m1, st1 = _mlp_stats(h1, agg1, W1a, b1a, W1b, b1b)
    out = _bn_elu_head(m1, st1, g1, be1, Wlin, blin)
    return out.reshape(-1)


# speedup vs baseline: 7.9328x; 1.0545x over previous
import functools

import jax
import jax.numpy as jnp
from jax import lax
from jax.experimental import pallas as pl
from jax.experimental.pallas import tpu as pltpu
from jax.experimental.pallas import tpu_sc as plsc

N_NODES = 10000
N_EDGES = 320000
EDGE_CHUNK = 80
NUM_WORKERS = 32
SUBCORES = 16


def _sc_segment_add(h, src2, dst2):
    n, f = h.shape
    nblk = src2.shape[0] // NUM_WORKERS
    sb = 25
    nsup = nblk // sb
    rz = n // SUBCORES
    nz8 = (n + 7) // 8
    mesh = plsc.VectorSubcoreMesh(core_axis_name="c", subcore_axis_name="s")

    @functools.partial(
        pl.kernel,
        mesh=mesh,
        out_type=jax.ShapeDtypeStruct((2, n, f), jnp.float32),
        compiler_params=pltpu.CompilerParams(use_tc_tiling_on_sc=False),
        scratch_types=[
            pltpu.VMEM_SHARED((n, f), jnp.float32),
            pltpu.VMEM((sb, EDGE_CHUNK), jnp.int32),
            pltpu.VMEM((sb, EDGE_CHUNK), jnp.int32),
            pltpu.VMEM((EDGE_CHUNK, f), jnp.float32),
            pltpu.VMEM((EDGE_CHUNK, f), jnp.float32),
            pltpu.VMEM((8, f), jnp.float32),
            pltpu.SemaphoreType.DMA,
            pltpu.SemaphoreType.DMA,
            pltpu.SemaphoreType.DMA,
            pltpu.SemaphoreType.DMA,
        ],
    )
    def k(h_hbm, src_hbm, dst_hbm, out_hbm, acc,
          src_v, dst_v, rows0, rows1, zbuf, g0, g1, s0, s1):
        c = lax.axis_index("c")
        s = lax.axis_index("s")
        wid = c * SUBCORES + s

        @pl.loop(0, 8)
        def _(r):
            @pl.loop(0, f // 16)
            def _(q):
                zbuf[r, pl.ds(q * 16, 16)] = jnp.zeros((16,), jnp.float32)

        @pl.loop(0, (nz8 + SUBCORES - 1) // SUBCORES)
        def _(i):
            blk = s + SUBCORES * i

            @pl.when(blk < nz8)
            def _():
                pltpu.sync_copy(zbuf, acc.at[pl.ds(blk * 8, 8)])

        plsc.subcore_barrier()

        def start_g(j, buf, sem):
            pltpu.async_copy(h_hbm.at[src_v.at[j]], buf, sem)

        def wait_g(j, buf, sem):
            pltpu.make_async_copy(h_hbm.at[src_v.at[j]], buf, sem).wait()

        def start_s(j, buf, sem):
            pltpu.async_copy(buf, acc.at[dst_v.at[j]], sem, add=True)

        def wait_s(j, buf, sem):
            pltpu.make_async_copy(buf, acc.at[dst_v.at[j]], sem).wait()

        @pl.loop(0, nsup)
        def _(t):
            base = wid * nblk + t * sb
            pltpu.sync_copy(src_hbm.at[pl.ds(base, sb)], src_v)
            pltpu.sync_copy(dst_hbm.at[pl.ds(base, sb)], dst_v)
            start_g(0, rows0, g0)

            @pl.loop(0, (sb - 1) // 2)
            def _(p):
                j0 = 2 * p
                wait_g(j0, rows0, g0)
                start_g(j0 + 1, rows1, g1)
                start_s(j0, rows0, s0)
                wait_g(j0 + 1, rows1, g1)
                wait_s(j0, rows0, s0)
                start_g(j0 + 2, rows0, g0)
                start_s(j0 + 1, rows1, s1)
                wait_s(j0 + 1, rows1, s1)

            wait_g(sb - 1, rows0, g0)
            pltpu.sync_copy(rows0, acc.at[dst_v.at[sb - 1]], add=True)

        plsc.subcore_barrier()
        pltpu.sync_copy(acc.at[pl.ds(s * rz, rz)], out_hbm.at[c, pl.ds(s * rz, rz)])

    return k(h, src2, dst2)


def _gin_dense(h, agg, Wa, ba, Wb, bb, g, be):
    n, f = h.shape
    hdim = Wb.shape[1]

    def body(h_ref, agg_ref, wa_ref, ba_ref, wb_ref, bb_ref, g_ref, be_ref,
             o_ref):
        m = h_ref[...] + agg_ref[0] + agg_ref[1]
        t = jnp.maximum(
            jnp.dot(m, wa_ref[...], preferred_element_type=jnp.float32)
            + ba_ref[...], 0.0)
        m2 = (jnp.dot(t, wb_ref[...], preferred_element_type=jnp.float32)
              + bb_ref[...])
        mean = jnp.mean(m2, axis=0, keepdims=True)
        var = jnp.mean(m2 * m2, axis=0, keepdims=True) - mean * mean
        scale = g_ref[...] * lax.rsqrt(var + 1e-5)
        shift = be_ref[...] - mean * scale
        v = m2 * scale + shift
        o_ref[...] = jnp.where(v > 0, v, jnp.exp(jnp.minimum(v, 0.0)) - 1.0)

    return pl.pallas_call(
        body,
        out_shape=jax.ShapeDtypeStruct((n, hdim), jnp.float32),
        compiler_params=pltpu.CompilerParams(
            vmem_limit_bytes=60 * 1024 * 1024),
    )(h, agg, Wa, ba.reshape(1, -1), Wb, bb.reshape(1, -1),
      g.reshape(1, -1), be.reshape(1, -1))


def _gin_dense_head(h, agg, Wa, ba, Wb, bb, g, be, Wlin, blin):
    n, f = h.shape
    hdim = Wb.shape[1]

    def body(h_ref, agg_ref, wa_ref, ba_ref, wb_ref, bb_ref, g_ref, be_ref,
             w_ref, b_ref, o_ref):
        m = h_ref[...] + agg_ref[0] + agg_ref[1]
        t = jnp.maximum(
            jnp.dot(m, wa_ref[...], preferred_element_type=jnp.float32)
            + ba_ref[...], 0.0)
        m2 = (jnp.dot(t, wb_ref[...], preferred_element_type=jnp.float32)
              + bb_ref[...])
        mean = jnp.mean(m2, axis=0, keepdims=True)
        var = jnp.mean(m2 * m2, axis=0, keepdims=True) - mean * mean
        scale = g_ref[...] * lax.rsqrt(var + 1e-5)
        shift = be_ref[...] - mean * scale
        v = m2 * scale + shift
        v = jnp.where(v > 0, v, jnp.exp(jnp.minimum(v, 0.0)) - 1.0)
        logit = jnp.sum(v * w_ref[...], axis=1, keepdims=True) + b_ref[...]
        o_ref[...] = 1.0 / (1.0 + jnp.exp(-logit))

    return pl.pallas_call(
        body,
        out_shape=jax.ShapeDtypeStruct((n, 1), jnp.float32),
        compiler_params=pltpu.CompilerParams(
            vmem_limit_bytes=60 * 1024 * 1024),
    )(h, agg, Wa, ba.reshape(1, -1), Wb, bb.reshape(1, -1),
      g.reshape(1, -1), be.reshape(1, -1),
      Wlin.reshape(1, -1), blin.reshape(1, 1))


def kernel(x, cdk_desc, edge_index, W0a, b0a, W0b, b0b, g0, be0,
           W1a, b1a, W1b, b1b, g1, be1, Wlin, blin):
    h0 = jnp.concatenate([x, cdk_desc], axis=-1)
    src2 = edge_index[0].reshape(N_EDGES // EDGE_CHUNK, EDGE_CHUNK)
    dst2 = edge_index[1].reshape(N_EDGES // EDGE_CHUNK, EDGE_CHUNK)

    agg0 = _sc_segment_add(h0, src2, dst2)
    h1 = _gin_dense(h0, agg0, W0a, b0a, W0b, b0b, g0, be0)

    agg1 = _sc_segment_add(h1, src2, dst2)
    out = _gin_dense_head(h1, agg1, W1a, b1a, W1b, b1b, g1, be1, Wlin, blin)
    return out.reshape(-1)


# speedup vs baseline: 8.9783x; 1.1318x over previous
import functools

import jax
import jax.numpy as jnp
from jax import lax
from jax.experimental import pallas as pl
from jax.experimental.pallas import tpu as pltpu
from jax.experimental.pallas import tpu_sc as plsc

N_NODES = 10000
N_EDGES = 320000
EDGE_CHUNK = 80
NUM_WORKERS = 32
SUBCORES = 16


def _sc_segment_add(h, src2, dst2):
    n, f = h.shape
    nblk = src2.shape[0] // NUM_WORKERS
    sb = 25
    nsup = nblk // sb
    rz = n // SUBCORES
    nz8 = (n + 7) // 8
    mesh = plsc.VectorSubcoreMesh(core_axis_name="c", subcore_axis_name="s")

    @functools.partial(
        pl.kernel,
        mesh=mesh,
        out_type=jax.ShapeDtypeStruct((2, n, f), jnp.float32),
        compiler_params=pltpu.CompilerParams(use_tc_tiling_on_sc=False),
        scratch_types=[
            pltpu.VMEM_SHARED((n, f), jnp.float32),
            pltpu.VMEM((sb, EDGE_CHUNK), jnp.int32),
            pltpu.VMEM((sb, EDGE_CHUNK), jnp.int32),
            pltpu.VMEM((EDGE_CHUNK, f), jnp.float32),
            pltpu.VMEM((EDGE_CHUNK, f), jnp.float32),
            pltpu.VMEM((8, f), jnp.float32),
            pltpu.SemaphoreType.DMA,
            pltpu.SemaphoreType.DMA,
            pltpu.SemaphoreType.DMA,
            pltpu.SemaphoreType.DMA,
        ],
    )
    def k(h_hbm, src_hbm, dst_hbm, out_hbm, acc,
          src_v, dst_v, rows0, rows1, zbuf, g0, g1, s0, s1):
        c = lax.axis_index("c")
        s = lax.axis_index("s")
        wid = c * SUBCORES + s

        @pl.loop(0, 8)
        def _(r):
            @pl.loop(0, f // 16)
            def _(q):
                zbuf[r, pl.ds(q * 16, 16)] = jnp.zeros((16,), jnp.float32)

        @pl.loop(0, (nz8 + SUBCORES - 1) // SUBCORES)
        def _(i):
            blk = s + SUBCORES * i

            @pl.when(blk < nz8)
            def _():
                pltpu.sync_copy(zbuf, acc.at[pl.ds(blk * 8, 8)])

        plsc.subcore_barrier()

        def start_g(j, buf, sem):
            pltpu.async_copy(h_hbm.at[src_v.at[j]], buf, sem)

        def wait_g(j, buf, sem):
            pltpu.make_async_copy(h_hbm.at[src_v.at[j]], buf, sem).wait()

        def start_s(j, buf, sem):
            pltpu.async_copy(buf, acc.at[dst_v.at[j]], sem, add=True)

        def wait_s(j, buf, sem):
            pltpu.make_async_copy(buf, acc.at[dst_v.at[j]], sem).wait()

        @pl.loop(0, nsup)
        def _(t):
            base = wid * nblk + t * sb
            pltpu.sync_copy(src_hbm.at[pl.ds(base, sb)], src_v)
            pltpu.sync_copy(dst_hbm.at[pl.ds(base, sb)], dst_v)
            start_g(0, rows0, g0)

            @pl.loop(0, (sb - 1) // 2)
            def _(p):
                j0 = 2 * p
                wait_g(j0, rows0, g0)
                start_g(j0 + 1, rows1, g1)
                start_s(j0, rows0, s0)
                wait_g(j0 + 1, rows1, g1)
                wait_s(j0, rows0, s0)
                start_g(j0 + 2, rows0, g0)
                start_s(j0 + 1, rows1, s1)
                wait_s(j0 + 1, rows1, s1)

            wait_g(sb - 1, rows0, g0)
            pltpu.sync_copy(rows0, acc.at[dst_v.at[sb - 1]], add=True)

        plsc.subcore_barrier()
        pltpu.sync_copy(acc.at[pl.ds(s * rz, rz)], out_hbm.at[c, pl.ds(s * rz, rz)])

    return k(h, src2, dst2)


_TC_PARAMS = pltpu.CompilerParams(vmem_limit_bytes=60 * 1024 * 1024)


def _project0(x, cdk, W0a):
    n, fx = x.shape
    fc = cdk.shape[1]
    hdim = W0a.shape[1]

    def body(x_ref, c_ref, wa_ref, o_ref):
        o_ref[...] = (
            jnp.dot(x_ref[...], wa_ref[0:fx, :],
                    preferred_element_type=jnp.float32)
            + jnp.dot(c_ref[...], wa_ref[fx:fx + fc, :],
                      preferred_element_type=jnp.float32))

    return pl.pallas_call(
        body,
        out_shape=jax.ShapeDtypeStruct((n, hdim), jnp.float32),
        compiler_params=_TC_PARAMS,
    )(x, cdk, W0a)


def _dense_mid(u, agg, ba, Wb, bb, g, be, Wnext):
    n, hdim = u.shape

    def body(u_ref, agg_ref, ba_ref, wb_ref, bb_ref, g_ref, be_ref, wn_ref,
             o_ref):
        t = jnp.maximum(u_ref[...] + agg_ref[0] + agg_ref[1] + ba_ref[...],
                        0.0)
        m2 = (jnp.dot(t, wb_ref[...], preferred_element_type=jnp.float32)
              + bb_ref[...])
        mean = jnp.mean(m2, axis=0, keepdims=True)
        var = jnp.mean(m2 * m2, axis=0, keepdims=True) - mean * mean
        scale = g_ref[...] * lax.rsqrt(var + 1e-5)
        shift = be_ref[...] - mean * scale
        v = m2 * scale + shift
        h1 = jnp.where(v > 0, v, jnp.exp(jnp.minimum(v, 0.0)) - 1.0)
        o_ref[...] = jnp.dot(h1, wn_ref[...],
                             preferred_element_type=jnp.float32)

    return pl.pallas_call(
        body,
        out_shape=jax.ShapeDtypeStruct((n, Wnext.shape[1]), jnp.float32),
        compiler_params=_TC_PARAMS,
    )(u, agg, ba.reshape(1, -1), Wb, bb.reshape(1, -1),
      g.reshape(1, -1), be.reshape(1, -1), Wnext)


def _dense_head(u, agg, ba, Wb, bb, g, be, Wlin, blin):
    n, hdim = u.shape

    def body(u_ref, agg_ref, ba_ref, wb_ref, bb_ref, g_ref, be_ref,
             w_ref, b_ref, o_ref):
        t = jnp.maximum(u_ref[...] + agg_ref[0] + agg_ref[1] + ba_ref[...],
                        0.0)
        m2 = (jnp.dot(t, wb_ref[...], preferred_element_type=jnp.float32)
              + bb_ref[...])
        mean = jnp.mean(m2, axis=0, keepdims=True)
        var = jnp.mean(m2 * m2, axis=0, keepdims=True) - mean * mean
        scale = g_ref[...] * lax.rsqrt(var + 1e-5)
        shift = be_ref[...] - mean * scale
        v = m2 * scale + shift
        v = jnp.where(v > 0, v, jnp.exp(jnp.minimum(v, 0.0)) - 1.0)
        logit = jnp.sum(v * w_ref[...], axis=1, keepdims=True) + b_ref[...]
        o_ref[...] = 1.0 / (1.0 + jnp.exp(-logit))

    return pl.pallas_call(
        body,
        out_shape=jax.ShapeDtypeStruct((n, 1), jnp.float32),
        compiler_params=_TC_PARAMS,
    )(u, agg, ba.reshape(1, -1), Wb, bb.reshape(1, -1),
      g.reshape(1, -1), be.reshape(1, -1),
      Wlin.reshape(1, -1), blin.reshape(1, 1))


def kernel(x, cdk_desc, edge_index, W0a, b0a, W0b, b0b, g0, be0,
           W1a, b1a, W1b, b1b, g1, be1, Wlin, blin):
    src2 = edge_index[0].reshape(N_EDGES // EDGE_CHUNK, EDGE_CHUNK)
    dst2 = edge_index[1].reshape(N_EDGES // EDGE_CHUNK, EDGE_CHUNK)

    u0 = _project0(x, cdk_desc, W0a)
    agg0 = _sc_segment_add(u0, src2, dst2)
    u1 = _dense_mid(u0, agg0, b0a, W0b, b0b, g0, be0, W1a)
    agg1 = _sc_segment_add(u1, src2, dst2)
    out = _dense_head(u1, agg1, b1a, W1b, b1b, g1, be1, Wlin, blin)
    return out.reshape(-1)


# speedup vs baseline: 11.2124x; 1.2488x over previous
import functools

import jax
import jax.numpy as jnp
from jax import lax
from jax.experimental import pallas as pl
from jax.experimental.pallas import tpu as pltpu
from jax.experimental.pallas import tpu_sc as plsc

N_NODES = 10000
N_EDGES = 320000
EDGE_CHUNK = 80
NUM_WORKERS = 32
SUBCORES = 16


def _sc_segment_add(h, src2, dst2):
    n, f = h.shape
    nblk = src2.shape[0] // NUM_WORKERS
    sb = 25
    nsup = nblk // sb
    rz = n // SUBCORES
    nz8 = (n + 7) // 8
    mesh = plsc.VectorSubcoreMesh(core_axis_name="c", subcore_axis_name="s")

    @functools.partial(
        pl.kernel,
        mesh=mesh,
        out_type=jax.ShapeDtypeStruct((2, n, f), jnp.float32),
        compiler_params=pltpu.CompilerParams(use_tc_tiling_on_sc=False),
        scratch_types=[
            pltpu.VMEM_SHARED((n, f), jnp.float32),
            pltpu.VMEM((sb, EDGE_CHUNK), jnp.int32),
            pltpu.VMEM((sb, EDGE_CHUNK), jnp.int32),
            pltpu.VMEM((EDGE_CHUNK, f), jnp.float32),
            pltpu.VMEM((EDGE_CHUNK, f), jnp.float32),
            pltpu.VMEM((EDGE_CHUNK, f), jnp.float32),
            pltpu.VMEM((EDGE_CHUNK, f), jnp.float32),
            pltpu.VMEM((8, f), jnp.float32),
            pltpu.SemaphoreType.DMA,
            pltpu.SemaphoreType.DMA,
            pltpu.SemaphoreType.DMA,
            pltpu.SemaphoreType.DMA,
            pltpu.SemaphoreType.DMA,
            pltpu.SemaphoreType.DMA,
            pltpu.SemaphoreType.DMA,
            pltpu.SemaphoreType.DMA,
        ],
    )
    def k(h_hbm, src_hbm, dst_hbm, out_hbm, acc,
          src_v, dst_v, rows0, rows1, rows2, rows3, zbuf,
          g0, g1, g2, g3, s0, s1, s2, s3):
        c = lax.axis_index("c")
        s = lax.axis_index("s")
        wid = c * SUBCORES + s

        @pl.loop(0, 8)
        def _(r):
            @pl.loop(0, f // 16)
            def _(q):
                zbuf[r, pl.ds(q * 16, 16)] = jnp.zeros((16,), jnp.float32)

        @pl.loop(0, (nz8 + SUBCORES - 1) // SUBCORES)
        def _(i):
            blk = s + SUBCORES * i

            @pl.when(blk < nz8)
            def _():
                pltpu.sync_copy(zbuf, acc.at[pl.ds(blk * 8, 8)])

        plsc.subcore_barrier()

        def start_g(j, buf, sem):
            pltpu.async_copy(h_hbm.at[src_v.at[j]], buf, sem)

        def wait_g(j, buf, sem):
            pltpu.make_async_copy(h_hbm.at[src_v.at[j]], buf, sem).wait()

        def start_s(j, buf, sem):
            pltpu.async_copy(buf, acc.at[dst_v.at[j]], sem, add=True)

        def wait_s(j, buf, sem):
            pltpu.make_async_copy(buf, acc.at[dst_v.at[j]], sem).wait()

        bufs = (rows0, rows1, rows2, rows3)
        gsems = (g0, g1, g2, g3)
        ssems = (s0, s1, s2, s3)

        @pl.loop(0, nsup)
        def _(t):
            base = wid * nblk + t * sb
            pltpu.sync_copy(src_hbm.at[pl.ds(base, sb)], src_v)
            pltpu.sync_copy(dst_hbm.at[pl.ds(base, sb)], dst_v)
            for q in range(4):
                start_g(q, bufs[q], gsems[q])

            @pl.loop(0, (sb - 1) // 4)
            def _(p):
                j0 = 4 * p
                for q in range(4):
                    wait_g(j0 + q, bufs[q], gsems[q])
                    start_s(j0 + q, bufs[q], ssems[q])
                for q in range(4):
                    wait_s(j0 + q, bufs[q], ssems[q])
                    jn = j0 + 4 + q

                    @pl.when(jn < sb)
                    def _():
                        start_g(jn, bufs[q], gsems[q])

            wait_g(sb - 1, rows0, g0)
            pltpu.sync_copy(rows0, acc.at[dst_v.at[sb - 1]], add=True)

        plsc.subcore_barrier()
        pltpu.sync_copy(acc.at[pl.ds(s * rz, rz)], out_hbm.at[c, pl.ds(s * rz, rz)])

    return k(h, src2, dst2)


_TC_PARAMS = pltpu.CompilerParams(vmem_limit_bytes=60 * 1024 * 1024)


def _project0(x, cdk, W0a):
    n, fx = x.shape
    fc = cdk.shape[1]
    hdim = W0a.shape[1]

    def body(x_ref, c_ref, wa_ref, o_ref):
        o_ref[...] = (
            jnp.dot(x_ref[...], wa_ref[0:fx, :],
                    preferred_element_type=jnp.float32)
            + jnp.dot(c_ref[...], wa_ref[fx:fx + fc, :],
                      preferred_element_type=jnp.float32))

    return pl.pallas_call(
        body,
        out_shape=jax.ShapeDtypeStruct((n, hdim), jnp.float32),
        compiler_params=_TC_PARAMS,
    )(x, cdk, W0a)


def _dense_mid(u, agg, ba, Wb, bb, g, be, Wnext):
    n, hdim = u.shape

    def body(u_ref, agg_ref, ba_ref, wb_ref, bb_ref, g_ref, be_ref, wn_ref,
             o_ref):
        t = jnp.maximum(u_ref[...] + agg_ref[0] + agg_ref[1] + ba_ref[...],
                        0.0)
        m2 = (jnp.dot(t, wb_ref[...], preferred_element_type=jnp.float32)
              + bb_ref[...])
        mean = jnp.mean(m2, axis=0, keepdims=True)
        var = jnp.mean(m2 * m2, axis=0, keepdims=True) - mean * mean
        scale = g_ref[...] * lax.rsqrt(var + 1e-5)
        shift = be_ref[...] - mean * scale
        v = m2 * scale + shift
        h1 = jnp.where(v > 0, v, jnp.exp(jnp.minimum(v, 0.0)) - 1.0)
        o_ref[...] = jnp.dot(h1, wn_ref[...],
                             preferred_element_type=jnp.float32)

    return pl.pallas_call(
        body,
        out_shape=jax.ShapeDtypeStruct((n, Wnext.shape[1]), jnp.float32),
        compiler_params=_TC_PARAMS,
    )(u, agg, ba.reshape(1, -1), Wb, bb.reshape(1, -1),
      g.reshape(1, -1), be.reshape(1, -1), Wnext)


def _dense_head(u, agg, ba, Wb, bb, g, be, Wlin, blin):
    n, hdim = u.shape

    def body(u_ref, agg_ref, ba_ref, wb_ref, bb_ref, g_ref, be_ref,
             w_ref, b_ref, o_ref):
        t = jnp.maximum(u_ref[...] + agg_ref[0] + agg_ref[1] + ba_ref[...],
                        0.0)
        m2 = (jnp.dot(t, wb_ref[...], preferred_element_type=jnp.float32)
              + bb_ref[...])
        mean = jnp.mean(m2, axis=0, keepdims=True)
        var = jnp.mean(m2 * m2, axis=0, keepdims=True) - mean * mean
        scale = g_ref[...] * lax.rsqrt(var + 1e-5)
        shift = be_ref[...] - mean * scale
        v = m2 * scale + shift
        v = jnp.where(v > 0, v, jnp.exp(jnp.minimum(v, 0.0)) - 1.0)
        logit = jnp.sum(v * w_ref[...], axis=1, keepdims=True) + b_ref[...]
        o_ref[...] = 1.0 / (1.0 + jnp.exp(-logit))

    return pl.pallas_call(
        body,
        out_shape=jax.ShapeDtypeStruct((n, 1), jnp.float32),
        compiler_params=_TC_PARAMS,
    )(u, agg, ba.reshape(1, -1), Wb, bb.reshape(1, -1),
      g.reshape(1, -1), be.reshape(1, -1),
      Wlin.reshape(1, -1), blin.reshape(1, 1))


def kernel(x, cdk_desc, edge_index, W0a, b0a, W0b, b0b, g0, be0,
           W1a, b1a, W1b, b1b, g1, be1, Wlin, blin):
    src2 = edge_index[0].reshape(N_EDGES // EDGE_CHUNK, EDGE_CHUNK)
    dst2 = edge_index[1].reshape(N_EDGES // EDGE_CHUNK, EDGE_CHUNK)

    u0 = _project0(x, cdk_desc, W0a)
    agg0 = _sc_segment_add(u0, src2, dst2)
    u1 = _dense_mid(u0, agg0, b0a, W0b, b0b, g0, be0, W1a)
    agg1 = _sc_segment_add(u1, src2, dst2)
    out = _dense_head(u1, agg1, b1a, W1b, b1b, g1, be1, Wlin, blin)
    return out.reshape(-1)
